# Initial kernel scaffold; baseline (speedup 1.0000x reference)
import jax
import jax.numpy as jnp
from jax.experimental import pallas as pl


def kernel(head_rep, tail_rep, tail_val, edge_list, rel_list, attn):
    raise NotImplementedError("write your pallas kernel here")


# speedup vs baseline: 10.6384x; 10.6384x over previous
import jax
import jax.numpy as jnp
from jax import lax
from jax.experimental import pallas as pl
from jax.experimental.pallas import tpu as pltpu
from jax.experimental.pallas import tpu_sc as plsc

N_NODES = 10000
NPAD = 10240
N_EDGES = 320000
D = 128
ALPHA = 0.2
NC, NS, L = 2, 16, 16
NW = NC * NS
EB = 128
N_BATCHES = N_EDGES // EB
TILE_ROWS = NPAD // NS
ROW_CHUNK = 128
N_CHUNKS = TILE_ROWS // ROW_CHUNK
FULL_I = N_BATCHES // NW
EXTRA = N_BATCHES - FULL_I * NW


def _scores_body(head_ref, tail_ref, attn_ref, ah_ref, at_ref):
    aw = attn_ref[...]
    ah_ref[...] = jnp.sum(head_ref[...] * aw[:, :D], axis=1, keepdims=True)
    at_ref[...] = jnp.sum(tail_ref[...] * aw[:, D:], axis=1, keepdims=True)


def _sc_body(ah_hbm, at_hbm, tv_hbm, src_hbm, dst_hbm,
             hp_out, rs_out,
             ah_tab, at_tab, src_buf, dst_buf, w_buf, rows,
             hp_acc, rs_acc, sem):
    c = lax.axis_index("c")
    s = lax.axis_index("s")
    wid = s * NC + c

    zero16 = jnp.zeros((L,), jnp.float32)

    def _zbody(r, carry):
        for j in range(D // L):
            rows[r, pl.ds(j * L, L)] = zero16
        return carry

    lax.fori_loop(0, ROW_CHUNK, _zbody, 0)

    tbase = s * TILE_ROWS
    for k in range(N_CHUNKS):
        pltpu.sync_copy(rows, hp_acc.at[pl.ds(tbase + k * ROW_CHUNK, ROW_CHUNK)])
        pltpu.sync_copy(rows.at[0], rs_acc.at[pl.ds(tbase + k * ROW_CHUNK, ROW_CHUNK)])

    pltpu.sync_copy(ah_hbm, ah_tab)
    pltpu.sync_copy(at_hbm, at_tab)

    plsc.subcore_barrier()

    n_i = FULL_I + jnp.where(wid < EXTRA, 1, 0)

    def _ebody(i, carry):
        base = (i * NW + wid) * EB
        base = pl.multiple_of(base, EB)
        pltpu.sync_copy(src_hbm.at[pl.ds(base, EB)], src_buf)
        pltpu.sync_copy(dst_hbm.at[pl.ds(base, EB)], dst_buf)
        pltpu.async_copy(tv_hbm.at[dst_buf], rows, sem).wait()
        for j in range(EB // L):
            si = src_buf[pl.ds(j * L, L)]
            di = dst_buf[pl.ds(j * L, L)]
            x = plsc.load_gather(ah_tab, [si]) + plsc.load_gather(at_tab, [di])
            x = jnp.clip(x, -10.0, 10.0)
            x = jnp.where(x >= 0.0, x, ALPHA * x)
            w_buf[pl.ds(j * L, L)] = jnp.exp(x)

        def _mbody(g, mcarry):
            wv = w_buf[pl.ds(g * L, L)]
            for l in range(L):
                wr = wv[l]
                r = g * L + l
                for j in range(D // L):
                    rows[r, pl.ds(j * L, L)] = rows[r, pl.ds(j * L, L)] * wr
            return mcarry

        lax.fori_loop(0, EB // L, _mbody, 0)
        pltpu.sync_copy(rows, hp_acc.at[src_buf], add=True)
        pltpu.sync_copy(w_buf, rs_acc.at[src_buf], add=True)
        return carry

    lax.fori_loop(0, n_i, _ebody, 0)

    plsc.subcore_barrier()

    pltpu.sync_copy(hp_acc.at[pl.ds(tbase, TILE_ROWS)],
                    hp_out.at[c, pl.ds(tbase, TILE_ROWS)])
    pltpu.sync_copy(rs_acc.at[pl.ds(tbase, TILE_ROWS)],
                    rs_out.at[c, pl.ds(tbase, TILE_ROWS)])


_CB = 1024


def _combine_body(hp_ref, rs_ref, hp_out_ref, rs_out_ref):
    hp_out_ref[...] = hp_ref[0] + hp_ref[1]
    rs_out_ref[...] = (rs_ref[0] + rs_ref[1])[:, None]


def kernel(head_rep, tail_rep, tail_val, edge_list, rel_list, attn):
    f32 = jnp.float32
    head_p = jnp.pad(head_rep.astype(f32), ((0, NPAD - N_NODES), (0, 0)))
    tail_p = jnp.pad(tail_rep.astype(f32), ((0, NPAD - N_NODES), (0, 0)))
    src = edge_list[0].astype(jnp.int32)
    dst = edge_list[1].astype(jnp.int32)

    ah2, at2 = pl.pallas_call(
        _scores_body,
        grid=(NPAD // _CB,),
        in_specs=[
            pl.BlockSpec((_CB, D), lambda i: (i, 0)),
            pl.BlockSpec((_CB, D), lambda i: (i, 0)),
            pl.BlockSpec((1, 2 * D), lambda i: (0, 0)),
        ],
        out_specs=[
            pl.BlockSpec((_CB, 1), lambda i: (i, 0)),
            pl.BlockSpec((_CB, 1), lambda i: (i, 0)),
        ],
        out_shape=[
            jax.ShapeDtypeStruct((NPAD, 1), f32),
            jax.ShapeDtypeStruct((NPAD, 1), f32),
        ],
    )(head_p, tail_p, attn.astype(f32))
    ah = ah2.reshape(NPAD)
    at = at2.reshape(NPAD)

    mesh = plsc.VectorSubcoreMesh(core_axis_name="c", subcore_axis_name="s")
    sc_fn = pl.kernel(
        _sc_body,
        mesh=mesh,
        compiler_params=pltpu.CompilerParams(needs_layout_passes=False),
        out_type=[
            jax.ShapeDtypeStruct((NC, NPAD, D), f32),
            jax.ShapeDtypeStruct((NC, NPAD), f32),
        ],
        scratch_types=[
            pltpu.VMEM((NPAD,), f32),
            pltpu.VMEM((NPAD,), f32),
            pltpu.VMEM((EB,), jnp.int32),
            pltpu.VMEM((EB,), jnp.int32),
            pltpu.VMEM((EB,), f32),
            pltpu.VMEM((EB, D), f32),
            pltpu.VMEM_SHARED((NPAD, D), f32),
            pltpu.VMEM_SHARED((NPAD,), f32),
            pltpu.SemaphoreType.DMA,
        ],
    )
    hp_part, rs_part = sc_fn(ah, at, tail_val.astype(f32), src, dst)

    hp, rs = pl.pallas_call(
        _combine_body,
        grid=(NPAD // _CB,),
        in_specs=[
            pl.BlockSpec((NC, _CB, D), lambda i: (0, i, 0)),
            pl.BlockSpec((NC, _CB), lambda i: (0, i)),
        ],
        out_specs=[
            pl.BlockSpec((_CB, D), lambda i: (i, 0)),
            pl.BlockSpec((_CB, 1), lambda i: (i, 0)),
        ],
        out_shape=[
            jax.ShapeDtypeStruct((N_NODES, D), f32),
            jax.ShapeDtypeStruct((N_NODES, 1), f32),
        ],
    )(hp_part, rs_part)

    return (rs, hp)
